# Initial kernel scaffold; baseline (speedup 1.0000x reference)
import jax
import jax.numpy as jnp
from jax.experimental import pallas as pl


def kernel(feats, indices, W, b, gamma, beta):
    raise NotImplementedError("write your pallas kernel here")


# speedup vs baseline: 2.3845x; 2.3845x over previous
import functools

import jax
import jax.numpy as jnp
from jax import lax
from jax.experimental import pallas as pl
from jax.experimental.pallas import tpu as pltpu
from jax.experimental.pallas import tpu_sc as plsc

_NSCALES = 4
_SHIFTS = (0, 3, 6, 9)
_C = 128

_NC = 2
_NS = 16
_NW = _NC * _NS
_CH = 80


def _morton_key(indices):
    b = indices[:, 0].astype(jnp.int32)
    cx = indices[:, 1].astype(jnp.int32) >> 1
    cy = indices[:, 2].astype(jnp.int32) >> 1
    cz = indices[:, 3].astype(jnp.int32) >> 1
    m = jnp.zeros_like(b)
    for k in range(6):
        m = m | (((cx >> k) & 1) << (3 * k + 2))
        m = m | (((cy >> k) & 1) << (3 * k + 1))
        m = m | (((cz >> k) & 1) << (3 * k))
    return (b << 18) | m


def _index_prep(indices, m, blk):
    nblk = m // blk
    key = _morton_key(indices)
    order = jnp.argsort(key).astype(jnp.int32)
    sk = jnp.take(key, order)
    iota = jnp.arange(m, dtype=jnp.int32)
    shifts = jnp.array(_SHIFTS, jnp.int32).reshape(_NSCALES, 1)
    ks = sk[None, :] >> shifts
    one_col = jnp.ones((_NSCALES, 1), jnp.bool_)
    bnd = jnp.concatenate([one_col, ks[:, 1:] != ks[:, :-1]], axis=1)
    seg = jnp.cumsum(bnd.astype(jnp.int32), axis=1) - 1
    islast = jnp.concatenate([bnd[:, 1:], one_col], axis=1)
    lastpos = jnp.flip(
        lax.cummin(jnp.flip(jnp.where(islast, iota[None, :], m), 1), axis=1), 1)
    local = seg - jnp.repeat(seg[:, ::blk], blk, axis=1)
    rowid = ((iota // blk) * blk)[None, :] + local
    gidx_sorted = jnp.take_along_axis(rowid, lastpos, axis=1)
    first_k = ks[:, ::blk]
    last_k = ks[:, blk - 1::blk]
    cont = jnp.concatenate(
        [jnp.zeros((_NSCALES, 1), jnp.int32),
         (first_k[:, 1:] == last_k[:, :-1]).astype(jnp.int32)], axis=1)
    lastloc = local[:, blk - 1::blk]
    per_scale = []
    for s in range(_NSCALES):
        per_scale.append({
            "gidx": gidx_sorted[s],
            "local": local[s].reshape(nblk, 1, blk),
            "cont": cont[s].reshape(nblk, 1, 1),
            "lastloc": lastloc[s].reshape(nblk, 1, 1),
        })
    return order, per_scale


_NB = 2


def _sc_gather(tables, gidxs, order=None):
    t_n = len(tables)
    m = gidxs[0].shape[0]
    pw = m // _NW
    nch = pw // _CH
    assert pw % _CH == 0 and m % _NW == 0
    nouter = (nch + _NB - 1) // _NB
    mesh = plsc.VectorSubcoreMesh(
        core_axis_name="c", subcore_axis_name="s", num_cores=_NC,
        num_subcores=_NS)
    out_type = [jax.ShapeDtypeStruct((m, _C), jnp.float32) for _ in range(t_n)]
    scratch = ([pltpu.VMEM((_CH,), jnp.int32) for _ in range(t_n * _NB)]
               + [pltpu.VMEM((_CH,), jnp.int32) for _ in range(_NB)]
               + [pltpu.VMEM((_CH, _C), jnp.float32)
                  for _ in range(t_n * _NB)]
               + [pltpu.SemaphoreType.DMA for _ in range(2 * _NB)])
    n_in = 2 * t_n + (0 if order is None else 1)

    @functools.partial(pl.kernel, out_type=out_type, mesh=mesh,
                       scratch_types=scratch)
    def gather_kernel(*refs):
        tbl = refs[:t_n]
        gix = refs[t_n:2 * t_n]
        oidx = None if order is None else refs[2 * t_n]
        outs = refs[n_in:n_in + t_n]
        sc = list(refs[n_in + t_n:])
        idxb = [sc[sl * t_n:(sl + 1) * t_n] for sl in range(_NB)]
        sc = sc[t_n * _NB:]
        ordb = sc[:_NB]
        sc = sc[_NB:]
        rowb = [sc[sl * t_n:(sl + 1) * t_n] for sl in range(_NB)]
        sc = sc[t_n * _NB:]
        sem_g = sc[:_NB]
        sem_s = sc[_NB:2 * _NB]
        w = lax.axis_index("s") * _NC + lax.axis_index("c")
        base = w * pw

        def dst(t, sl, off):
            if order is None:
                return outs[t].at[pl.ds(off, _CH)]
            return outs[t].at[ordb[sl]]

        def outer(g, carry):
            for sl in range(_NB):
                j = g * _NB + sl

                @pl.when(j < nch)
                def _():
                    off = base + j * _CH

                    @pl.when(j >= _NB)
                    def _():
                        poff = base + (j - _NB) * _CH
                        for t in range(t_n):
                            pltpu.make_async_copy(
                                rowb[sl][t], dst(t, sl, poff),
                                sem_s[sl]).wait()

                    if order is not None:
                        pltpu.sync_copy(oidx.at[pl.ds(off, _CH)], ordb[sl])
                    for t in range(t_n):
                        pltpu.sync_copy(gix[t].at[pl.ds(off, _CH)],
                                        idxb[sl][t])
                        pltpu.async_copy(tbl[t].at[idxb[sl][t]],
                                         rowb[sl][t], sem_g[sl])
                    for t in range(t_n):
                        pltpu.make_async_copy(tbl[t].at[idxb[sl][t]],
                                              rowb[sl][t], sem_g[sl]).wait()
                    for t in range(t_n):
                        pltpu.async_copy(rowb[sl][t], dst(t, sl, off),
                                         sem_s[sl])
            return carry

        lax.fori_loop(0, nouter, outer, 0)
        for sl in range(_NB):
            j_last = ((nch - 1 - sl) // _NB) * _NB + sl
            if j_last >= 0:
                off = base + j_last * _CH
                for t in range(t_n):
                    pltpu.make_async_copy(rowb[sl][t], dst(t, sl, off),
                                          sem_s[sl]).wait()

    args = list(tables) + list(gidxs)
    if order is not None:
        args.append(order)
    res = gather_kernel(*args)
    return res if isinstance(res, (list, tuple)) else [res]


def _segmean_body(blk, feats_ref, l0, l1, l2, l3, c0, c1, c2, c3,
                  ll0, ll1, ll2, ll3, o0, o1, o2, o3, carry_ref):
    f = feats_ref[...]
    lrefs = (l0, l1, l2, l3)
    crefs = (c0, c1, c2, c3)
    llrefs = (ll0, ll1, ll2, ll3)
    orefs = (o0, o1, o2, o3)
    row_iota = lax.broadcasted_iota(jnp.int32, (blk, _C), 0)
    oh_iota = lax.broadcasted_iota(jnp.int32, (blk, blk), 0)
    f1 = f.astype(jnp.bfloat16)
    f2 = (f - f1.astype(jnp.float32)).astype(jnp.bfloat16)
    rhs = jnp.concatenate(
        [f1, f2, jnp.ones((blk, _C), jnp.bfloat16)], axis=1)
    dn = (((1,), (0,)), ((), ()))
    for s in range(_NSCALES):
        lids = lrefs[s][0]
        oh = (oh_iota == lids).astype(jnp.bfloat16)
        big = lax.dot_general(oh, rhs, dn,
                              preferred_element_type=jnp.float32)
        sums = big[:, :_C] + big[:, _C:2 * _C]
        cnts = big[:, 2 * _C:]
        cont = crefs[s][0, 0, 0]
        use_carry = jnp.logical_and(row_iota == 0, cont == 1)
        sums = sums + jnp.where(use_carry, carry_ref[s, 0:1, :], 0.0)
        cnts = cnts + jnp.where(use_carry, carry_ref[s, 1:2, :], 0.0)
        ll = llrefs[s][0, 0, 0]
        sel = row_iota == ll
        carry_ref[s, 0:1, :] = jnp.sum(jnp.where(sel, sums, 0.0),
                                       axis=0, keepdims=True)
        carry_ref[s, 1:2, :] = jnp.sum(jnp.where(sel, cnts, 0.0),
                                       axis=0, keepdims=True)
        orefs[s][...] = sums / jnp.maximum(cnts, 1.0)


def _segment_means(feats_sorted, per_scale, m, blk, interpret=False):
    nblk = m // blk
    grid = (nblk,)
    in_specs = [pl.BlockSpec((blk, _C), lambda k: (k, 0))]
    in_specs += [pl.BlockSpec((1, 1, blk), lambda k: (k, 0, 0))
                 for _ in range(_NSCALES)]
    in_specs += [pl.BlockSpec((1, 1, 1), lambda k: (k, 0, 0),
                              memory_space=pltpu.SMEM)
                 for _ in range(2 * _NSCALES)]
    out_specs = [pl.BlockSpec((blk, _C), lambda k: (k, 0))
                 for _ in range(_NSCALES)]
    out_shape = [jax.ShapeDtypeStruct((m, _C), jnp.float32)
                 for _ in range(_NSCALES)]
    args = ([feats_sorted] + [d["local"] for d in per_scale]
            + [d["cont"] for d in per_scale]
            + [d["lastloc"] for d in per_scale])
    return pl.pallas_call(
        functools.partial(_segmean_body, blk),
        grid=grid,
        in_specs=in_specs,
        out_specs=out_specs,
        out_shape=out_shape,
        scratch_shapes=[pltpu.VMEM((_NSCALES, 8, _C), jnp.float32)],
        interpret=interpret,
    )(*args)


def _stats_body(f_ref, v0, v1, v2, v3, w_ref, b_ref, stat_ref, acc_ref):
    k = pl.program_id(0)

    @pl.when(k == 0)
    def _():
        acc_ref[...] = jnp.zeros_like(acc_ref)

    f = f_ref[...]
    vrefs = (v0, v1, v2, v3)
    for s in range(_NSCALES):
        v = vrefs[s][...]
        os_ = (f - v) * f
        h = lax.dot_general(os_, w_ref[s], (((1,), (0,)), ((), ())),
                            preferred_element_type=jnp.float32)
        h = h + b_ref[s : s + 1, :]
        acc_ref[s, 0:1, :] += jnp.sum(h, axis=0, keepdims=True)
        acc_ref[s, 1:2, :] += jnp.sum(h * h, axis=0, keepdims=True)
    stat_ref[...] = acc_ref[...]


def _proj_stats(feats, vups, w, b, m, blk, interpret=False):
    nblk = m // blk
    in_specs = [pl.BlockSpec((blk, _C), lambda k: (k, 0))]
    in_specs += [pl.BlockSpec((blk, _C), lambda k: (k, 0))
                 for _ in range(_NSCALES)]
    in_specs += [pl.BlockSpec((_NSCALES, _C, _C), lambda k: (0, 0, 0)),
                 pl.BlockSpec((_NSCALES, _C), lambda k: (0, 0))]
    return pl.pallas_call(
        _stats_body,
        grid=(nblk,),
        in_specs=in_specs,
        out_specs=pl.BlockSpec((_NSCALES, 8, _C), lambda k: (0, 0, 0)),
        out_shape=jax.ShapeDtypeStruct((_NSCALES, 8, _C), jnp.float32),
        scratch_shapes=[pltpu.VMEM((_NSCALES, 8, _C), jnp.float32)],
        interpret=interpret,
    )(feats, *vups, w, b)


def _emit_body(f_ref, v0, v1, v2, v3, w_ref, b_ref, sc_ref, sh_ref, out_ref):
    f = f_ref[...]
    vrefs = (v0, v1, v2, v3)
    for s in range(_NSCALES):
        v = vrefs[s][...]
        os_ = (f - v) * f
        h = lax.dot_general(os_, w_ref[s], (((1,), (0,)), ((), ())),
                            preferred_element_type=jnp.float32)
        h = h + b_ref[s : s + 1, :]
        y = h * sc_ref[s : s + 1, :] + sh_ref[s : s + 1, :]
        out_ref[:, s, :] = jnp.where(y >= 0, y, 0.01 * y)


def _proj_emit(feats, vups, w, b, scale, shift, m, blk, interpret=False):
    nblk = m // blk
    in_specs = [pl.BlockSpec((blk, _C), lambda k: (k, 0))]
    in_specs += [pl.BlockSpec((blk, _C), lambda k: (k, 0))
                 for _ in range(_NSCALES)]
    in_specs += [pl.BlockSpec((_NSCALES, _C, _C), lambda k: (0, 0, 0)),
                 pl.BlockSpec((_NSCALES, _C), lambda k: (0, 0)),
                 pl.BlockSpec((_NSCALES, _C), lambda k: (0, 0)),
                 pl.BlockSpec((_NSCALES, _C), lambda k: (0, 0))]
    return pl.pallas_call(
        _emit_body,
        grid=(nblk,),
        in_specs=in_specs,
        out_specs=pl.BlockSpec((blk, _NSCALES, _C), lambda k: (k, 0, 0)),
        out_shape=jax.ShapeDtypeStruct((m, _NSCALES, _C), jnp.float32),
        interpret=interpret,
    )(feats, *vups, w, b, scale, shift)


def kernel(feats, indices, W, b, gamma, beta):
    m = feats.shape[0]
    blk = 640
    order, per_scale = _index_prep(indices, m, blk)

    (feats_sorted,) = _sc_gather([feats], [order])
    means = _segment_means(feats_sorted, per_scale, m, blk)
    vups = _sc_gather(list(means), [d["gidx"] for d in per_scale],
                      order=order)
    stats = _proj_stats(feats, vups, W, b, m, blk)

    sumh = stats[:, 0, :]
    sumh2 = stats[:, 1, :]
    mean = sumh / m
    var = sumh2 / m - mean * mean
    scale = gamma / jnp.sqrt(var + 1e-5)
    shift = beta - mean * scale
    return _proj_emit(feats, vups, W, b, scale, shift, m, blk)


# speedup vs baseline: 2.4238x; 1.0165x over previous
import functools

import jax
import jax.numpy as jnp
from jax import lax
from jax.experimental import pallas as pl
from jax.experimental.pallas import tpu as pltpu
from jax.experimental.pallas import tpu_sc as plsc

_NSCALES = 4
_SHIFTS = (0, 3, 6, 9)
_C = 128

_NC = 2
_NS = 16
_NW = _NC * _NS
_CH = 80


def _morton_key(indices):
    b = indices[:, 0].astype(jnp.int32)
    cx = indices[:, 1].astype(jnp.int32) >> 1
    cy = indices[:, 2].astype(jnp.int32) >> 1
    cz = indices[:, 3].astype(jnp.int32) >> 1
    m = jnp.zeros_like(b)
    for k in range(6):
        m = m | (((cx >> k) & 1) << (3 * k + 2))
        m = m | (((cy >> k) & 1) << (3 * k + 1))
        m = m | (((cz >> k) & 1) << (3 * k))
    return (b << 18) | m


def _index_prep(indices, m, blk):
    nblk = m // blk
    key = _morton_key(indices)
    order = jnp.argsort(key).astype(jnp.int32)
    sk = jnp.take(key, order)
    iota = jnp.arange(m, dtype=jnp.int32)
    shifts = jnp.array(_SHIFTS, jnp.int32).reshape(_NSCALES, 1)
    ks = sk[None, :] >> shifts
    one_col = jnp.ones((_NSCALES, 1), jnp.bool_)
    bnd = jnp.concatenate([one_col, ks[:, 1:] != ks[:, :-1]], axis=1)
    seg = jnp.cumsum(bnd.astype(jnp.int32), axis=1) - 1
    islast = jnp.concatenate([bnd[:, 1:], one_col], axis=1)
    lastpos = jnp.flip(
        lax.cummin(jnp.flip(jnp.where(islast, iota[None, :], m), 1), axis=1), 1)
    local = seg - jnp.repeat(seg[:, ::blk], blk, axis=1)
    rowid = ((iota // blk) * blk)[None, :] + local
    gidx_sorted = jnp.take_along_axis(rowid, lastpos, axis=1)
    first_k = ks[:, ::blk]
    last_k = ks[:, blk - 1::blk]
    cont = jnp.concatenate(
        [jnp.zeros((_NSCALES, 1), jnp.int32),
         (first_k[:, 1:] == last_k[:, :-1]).astype(jnp.int32)], axis=1)
    lastloc = local[:, blk - 1::blk]
    per_scale = []
    for s in range(_NSCALES):
        per_scale.append({
            "gidx": gidx_sorted[s],
            "local": local[s].reshape(nblk, 1, blk),
            "cont": cont[s].reshape(nblk, 1, 1),
            "lastloc": lastloc[s].reshape(nblk, 1, 1),
        })
    return order, per_scale


_NB = 2


def _sc_gather(tables, gidxs, order=None):
    t_n = len(tables)
    m = gidxs[0].shape[0]
    pw = m // _NW
    nch = pw // _CH
    assert pw % _CH == 0 and m % _NW == 0
    nouter = (nch + _NB - 1) // _NB
    mesh = plsc.VectorSubcoreMesh(
        core_axis_name="c", subcore_axis_name="s", num_cores=_NC,
        num_subcores=_NS)
    out_type = [jax.ShapeDtypeStruct((m, _C), jnp.float32) for _ in range(t_n)]
    scratch = ([pltpu.VMEM((_CH,), jnp.int32) for _ in range(t_n * _NB)]
               + [pltpu.VMEM((_CH,), jnp.int32) for _ in range(_NB)]
               + [pltpu.VMEM((_CH, _C), jnp.float32)
                  for _ in range(t_n * _NB)]
               + [pltpu.SemaphoreType.DMA for _ in range(2 * _NB)])
    n_in = 2 * t_n + (0 if order is None else 1)

    @functools.partial(pl.kernel, out_type=out_type, mesh=mesh,
                       scratch_types=scratch)
    def gather_kernel(*refs):
        tbl = refs[:t_n]
        gix = refs[t_n:2 * t_n]
        oidx = None if order is None else refs[2 * t_n]
        outs = refs[n_in:n_in + t_n]
        sc = list(refs[n_in + t_n:])
        idxb = [sc[sl * t_n:(sl + 1) * t_n] for sl in range(_NB)]
        sc = sc[t_n * _NB:]
        ordb = sc[:_NB]
        sc = sc[_NB:]
        rowb = [sc[sl * t_n:(sl + 1) * t_n] for sl in range(_NB)]
        sc = sc[t_n * _NB:]
        sem_g = sc[:_NB]
        sem_s = sc[_NB:2 * _NB]
        w = lax.axis_index("s") * _NC + lax.axis_index("c")
        base = w * pw

        def dst(t, sl, off):
            if order is None:
                return outs[t].at[pl.ds(off, _CH)]
            return outs[t].at[ordb[sl]]

        def outer(g, carry):
            for sl in range(_NB):
                j = g * _NB + sl

                @pl.when(j < nch)
                def _():
                    off = base + j * _CH

                    @pl.when(j >= _NB)
                    def _():
                        poff = base + (j - _NB) * _CH
                        for t in range(t_n):
                            pltpu.make_async_copy(
                                rowb[sl][t], dst(t, sl, poff),
                                sem_s[sl]).wait()

                    if order is not None:
                        pltpu.sync_copy(oidx.at[pl.ds(off, _CH)], ordb[sl])
                    for t in range(t_n):
                        pltpu.sync_copy(gix[t].at[pl.ds(off, _CH)],
                                        idxb[sl][t])
                        pltpu.async_copy(tbl[t].at[idxb[sl][t]],
                                         rowb[sl][t], sem_g[sl])
                    for t in range(t_n):
                        pltpu.make_async_copy(tbl[t].at[idxb[sl][t]],
                                              rowb[sl][t], sem_g[sl]).wait()
                    for t in range(t_n):
                        pltpu.async_copy(rowb[sl][t], dst(t, sl, off),
                                         sem_s[sl])
            return carry

        lax.fori_loop(0, nouter, outer, 0)
        for sl in range(_NB):
            j_last = ((nch - 1 - sl) // _NB) * _NB + sl
            if j_last >= 0:
                off = base + j_last * _CH
                for t in range(t_n):
                    pltpu.make_async_copy(rowb[sl][t], dst(t, sl, off),
                                          sem_s[sl]).wait()

    args = list(tables) + list(gidxs)
    if order is not None:
        args.append(order)
    res = gather_kernel(*args)
    return res if isinstance(res, (list, tuple)) else [res]


def _segmean_body(blk, feats_ref, l0, l1, l2, l3, c0, c1, c2, c3,
                  ll0, ll1, ll2, ll3, o0, o1, o2, o3, carry_ref):
    f = feats_ref[...]
    lrefs = (l0, l1, l2, l3)
    crefs = (c0, c1, c2, c3)
    llrefs = (ll0, ll1, ll2, ll3)
    orefs = (o0, o1, o2, o3)
    row_iota = lax.broadcasted_iota(jnp.int32, (blk, _C), 0)
    oh_iota = lax.broadcasted_iota(jnp.int32, (blk, blk), 0)
    f1 = f.astype(jnp.bfloat16)
    f2 = (f - f1.astype(jnp.float32)).astype(jnp.bfloat16)
    rhs = jnp.concatenate(
        [f1, f2, jnp.ones((blk, _C), jnp.bfloat16)], axis=1)
    dn = (((1,), (0,)), ((), ()))
    for s in range(_NSCALES):
        lids = lrefs[s][0]
        oh = (oh_iota == lids).astype(jnp.bfloat16)
        big = lax.dot_general(oh, rhs, dn,
                              preferred_element_type=jnp.float32)
        sums = big[:, :_C] + big[:, _C:2 * _C]
        cnts = big[:, 2 * _C:]
        cont = crefs[s][0, 0, 0]
        use_carry = jnp.logical_and(row_iota == 0, cont == 1)
        sums = sums + jnp.where(use_carry, carry_ref[s, 0:1, :], 0.0)
        cnts = cnts + jnp.where(use_carry, carry_ref[s, 1:2, :], 0.0)
        ll = llrefs[s][0, 0, 0]
        sel = row_iota == ll
        carry_ref[s, 0:1, :] = jnp.sum(jnp.where(sel, sums, 0.0),
                                       axis=0, keepdims=True)
        carry_ref[s, 1:2, :] = jnp.sum(jnp.where(sel, cnts, 0.0),
                                       axis=0, keepdims=True)
        orefs[s][...] = sums / jnp.maximum(cnts, 1.0)


def _segment_means(feats_sorted, per_scale, m, blk, interpret=False):
    nblk = m // blk
    grid = (nblk,)
    in_specs = [pl.BlockSpec((blk, _C), lambda k: (k, 0))]
    in_specs += [pl.BlockSpec((1, 1, blk), lambda k: (k, 0, 0))
                 for _ in range(_NSCALES)]
    in_specs += [pl.BlockSpec((1, 1, 1), lambda k: (k, 0, 0),
                              memory_space=pltpu.SMEM)
                 for _ in range(2 * _NSCALES)]
    out_specs = [pl.BlockSpec((blk, _C), lambda k: (k, 0))
                 for _ in range(_NSCALES)]
    out_shape = [jax.ShapeDtypeStruct((m, _C), jnp.float32)
                 for _ in range(_NSCALES)]
    args = ([feats_sorted] + [d["local"] for d in per_scale]
            + [d["cont"] for d in per_scale]
            + [d["lastloc"] for d in per_scale])
    return pl.pallas_call(
        functools.partial(_segmean_body, blk),
        grid=grid,
        in_specs=in_specs,
        out_specs=out_specs,
        out_shape=out_shape,
        scratch_shapes=[pltpu.VMEM((_NSCALES, 8, _C), jnp.float32)],
        interpret=interpret,
    )(*args)


def _stats_body(f_ref, v0, v1, v2, v3, w_ref, b_ref, stat_ref, acc_ref):
    k = pl.program_id(0)

    @pl.when(k == 0)
    def _():
        acc_ref[...] = jnp.zeros_like(acc_ref)

    f = f_ref[...]
    vrefs = (v0, v1, v2, v3)
    for s in range(_NSCALES):
        v = vrefs[s][...]
        os_ = (f - v) * f
        h = lax.dot_general(os_, w_ref[s], (((1,), (0,)), ((), ())),
                            preferred_element_type=jnp.float32)
        h = h + b_ref[s : s + 1, :]
        acc_ref[s, 0:1, :] += jnp.sum(h, axis=0, keepdims=True)
        acc_ref[s, 1:2, :] += jnp.sum(h * h, axis=0, keepdims=True)
    stat_ref[...] = acc_ref[...]


def _proj_stats(feats, vups, w, b, m, blk, interpret=False):
    nblk = m // blk
    in_specs = [pl.BlockSpec((blk, _C), lambda k: (k, 0))]
    in_specs += [pl.BlockSpec((blk, _C), lambda k: (k, 0))
                 for _ in range(_NSCALES)]
    in_specs += [pl.BlockSpec((_NSCALES, _C, _C), lambda k: (0, 0, 0)),
                 pl.BlockSpec((_NSCALES, _C), lambda k: (0, 0))]
    return pl.pallas_call(
        _stats_body,
        grid=(nblk,),
        in_specs=in_specs,
        out_specs=pl.BlockSpec((_NSCALES, 8, _C), lambda k: (0, 0, 0)),
        out_shape=jax.ShapeDtypeStruct((_NSCALES, 8, _C), jnp.float32),
        scratch_shapes=[pltpu.VMEM((_NSCALES, 8, _C), jnp.float32)],
        interpret=interpret,
    )(feats, *vups, w, b)


def _emit_body(f_ref, v0, v1, v2, v3, w_ref, b_ref, sc_ref, sh_ref, out_ref):
    f = f_ref[...]
    vrefs = (v0, v1, v2, v3)
    for s in range(_NSCALES):
        v = vrefs[s][...]
        os_ = (f - v) * f
        h = lax.dot_general(os_, w_ref[s], (((1,), (0,)), ((), ())),
                            preferred_element_type=jnp.float32)
        h = h + b_ref[s : s + 1, :]
        y = h * sc_ref[s : s + 1, :] + sh_ref[s : s + 1, :]
        out_ref[:, s, :] = jnp.where(y >= 0, y, 0.01 * y)


def _proj_emit(feats, vups, w, b, scale, shift, m, blk, interpret=False):
    nblk = m // blk
    in_specs = [pl.BlockSpec((blk, _C), lambda k: (k, 0))]
    in_specs += [pl.BlockSpec((blk, _C), lambda k: (k, 0))
                 for _ in range(_NSCALES)]
    in_specs += [pl.BlockSpec((_NSCALES, _C, _C), lambda k: (0, 0, 0)),
                 pl.BlockSpec((_NSCALES, _C), lambda k: (0, 0)),
                 pl.BlockSpec((_NSCALES, _C), lambda k: (0, 0)),
                 pl.BlockSpec((_NSCALES, _C), lambda k: (0, 0))]
    return pl.pallas_call(
        _emit_body,
        grid=(nblk,),
        in_specs=in_specs,
        out_specs=pl.BlockSpec((blk, _NSCALES, _C), lambda k: (k, 0, 0)),
        out_shape=jax.ShapeDtypeStruct((m, _NSCALES, _C), jnp.float32),
        interpret=interpret,
    )(feats, *vups, w, b, scale, shift)


def kernel(feats, indices, W, b, gamma, beta):
    m = feats.shape[0]
    blk = 512
    order, per_scale = _index_prep(indices, m, blk)

    (feats_sorted,) = _sc_gather([feats], [order])
    means = _segment_means(feats_sorted, per_scale, m, blk)
    vups = _sc_gather(list(means), [d["gidx"] for d in per_scale],
                      order=order)
    stats = _proj_stats(feats, vups, W, b, m, blk)

    sumh = stats[:, 0, :]
    sumh2 = stats[:, 1, :]
    mean = sumh / m
    var = sumh2 / m - mean * mean
    scale = gamma / jnp.sqrt(var + 1e-5)
    shift = beta - mean * scale
    return _proj_emit(feats, vups, W, b, scale, shift, m, blk)
